# baseline (speedup 1.0000x reference)
import functools

import jax
import jax.numpy as jnp
from jax import lax
from jax.experimental import pallas as pl
from jax.experimental.pallas import tpu as pltpu
from jax.experimental.pallas import tpu_sc as plsc

ROWS = 128
COLS = 32768
NC = 2
NS = 16
L = 16
NW = NC * NS
CPW = COLS // NW
CHUNK = 256
NCHUNK = CPW // CHUNK
G = CHUNK // L
GU = 2


def _sc_body(x_hbm, out_hbm, buf0, buf1, idx_v, sem0, sem1):
    wid = lax.axis_index("s") * NC + lax.axis_index("c")
    base = wid * CPW
    bufs = (buf0, buf1)
    sems = (sem0, sem1)

    def src(ci):
        return x_hbm.at[:, pl.ds(base + ci * CHUNK, CHUNK)]

    copies = [None] * NCHUNK
    copies[0] = pltpu.async_copy(src(0), bufs[0], sems[0])
    for ci in range(NCHUNK):
        if ci + 1 < NCHUNK:
            copies[ci + 1] = pltpu.async_copy(
                src(ci + 1), bufs[(ci + 1) % 2], sems[(ci + 1) % 2])
        copies[ci].wait()
        buf = bufs[ci % 2]

        def g_step(gi, acc, buf=buf, ci=ci):
            for u in range(GU):
                col = (gi * GU + u) * L
                mv = jnp.full((L,), -jnp.inf, jnp.float32)
                mi = jnp.zeros((L,), jnp.int32)
                for r in range(0, ROWS, 2):
                    v0 = buf[r, pl.ds(col, L)]
                    v1 = buf[r + 1, pl.ds(col, L)]
                    g01 = v1 > v0
                    vp = jnp.maximum(v0, v1)
                    ip = jnp.where(g01, jnp.full((L,), r + 1, jnp.int32),
                                   jnp.full((L,), r, jnp.int32))
                    gt = vp > mv
                    mv = jnp.maximum(mv, vp)
                    mi = jnp.where(gt, ip, mi)
                idx_v[pl.ds(ci * CHUNK + col, L)] = mi
            return acc

        lax.fori_loop(0, G // GU, g_step, jnp.int32(0))

    pltpu.sync_copy(idx_v, out_hbm.at[pl.ds(base, CPW)])


@jax.jit
def _argmax_sc(x):
    mesh = plsc.VectorSubcoreMesh(core_axis_name="c", subcore_axis_name="s")
    f = pl.kernel(
        _sc_body,
        out_type=jax.ShapeDtypeStruct((COLS,), jnp.int32),
        mesh=mesh,
        scratch_types=[
            pltpu.VMEM((ROWS, CHUNK), jnp.float32),
            pltpu.VMEM((ROWS, CHUNK), jnp.float32),
            pltpu.VMEM((CPW,), jnp.int32),
            pltpu.SemaphoreType.DMA,
            pltpu.SemaphoreType.DMA,
        ],
    )
    return f(x)


def kernel(x):
    return _argmax_sc(x).reshape(1, COLS).astype(jnp.int64)


# speedup vs baseline: 2.5356x; 2.5356x over previous
import jax
import jax.numpy as jnp
from jax import lax
from jax.experimental import pallas as pl
from jax.experimental.pallas import tpu as pltpu

ROWS = 128
COLS = 32768
BW = 2048
GRID = COLS // BW


def _tc_body(x_ref, o_ref):
    v = x_ref[...]
    ridx = lax.broadcasted_iota(jnp.int32, (ROWS, BW), 0)
    mx = jnp.max(v, axis=0, keepdims=True)
    cand = jnp.where(v == mx, ridx, jnp.int32(ROWS))
    o_ref[...] = jnp.min(cand, axis=0, keepdims=True)


@jax.jit
def _argmax_tc(x):
    return pl.pallas_call(
        _tc_body,
        grid=(GRID,),
        in_specs=[pl.BlockSpec((ROWS, BW), lambda i: (0, i))],
        out_specs=pl.BlockSpec((1, BW), lambda i: (0, i)),
        out_shape=jax.ShapeDtypeStruct((1, COLS), jnp.int32),
    )(x)


def kernel(x):
    return _argmax_tc(x).astype(jnp.int64)
